# baseline (speedup 1.0000x reference)
import jax
import jax.numpy as jnp
from jax.experimental import pallas as pl
from jax.experimental.pallas import tpu as pltpu

_NCHUNK = 2
_NSUB = 4


def _mm_manual(x_hbm, w_ref, o_hbm, xbuf, obuf, insem, outsem):
    n, c = xbuf.shape[0], xbuf.shape[1]
    s = c // _NSUB
    for i in range(n):
        pltpu.make_async_copy(
            x_hbm.at[pl.ds(i * c, c), :], xbuf.at[i], insem.at[i]
        ).start()
    for i in range(n):
        pltpu.make_async_copy(
            x_hbm.at[pl.ds(i * c, c), :], xbuf.at[i], insem.at[i]
        ).wait()
        for j in range(_NSUB):
            k = i * _NSUB + j
            obuf[k] = jax.lax.dot_general(
                xbuf[i, pl.ds(j * s, s), :], w_ref[...],
                dimension_numbers=(((1,), (1,)), ((), ())),
                preferred_element_type=jnp.float32,
            )
            pltpu.make_async_copy(
                obuf.at[k], o_hbm.at[pl.ds(k * s, s), :], outsem.at[k]
            ).start()
    for k in range(n * _NSUB):
        s0 = o_hbm.shape[0] // (n * _NSUB)
        pltpu.make_async_copy(
            obuf.at[k], o_hbm.at[pl.ds(k * s0, s0), :], outsem.at[k]
        ).wait()


def kernel(x, W):
    B, S, D = x.shape
    M = B * S
    c = M // _NCHUNK
    s = c // _NSUB
    x2 = x.reshape(M, D)
    out = pl.pallas_call(
        _mm_manual,
        in_specs=[
            pl.BlockSpec(memory_space=pltpu.MemorySpace.HBM),
            pl.BlockSpec(memory_space=pltpu.MemorySpace.VMEM),
        ],
        out_specs=pl.BlockSpec(memory_space=pltpu.MemorySpace.HBM),
        out_shape=jax.ShapeDtypeStruct((M, D), jnp.float32),
        scratch_shapes=[
            pltpu.VMEM((_NCHUNK, c, D), jnp.float32),
            pltpu.VMEM((_NCHUNK * _NSUB, s, D), jnp.float32),
            pltpu.SemaphoreType.DMA((_NCHUNK,)),
            pltpu.SemaphoreType.DMA((_NCHUNK * _NSUB,)),
        ],
    )(x2, W)
    return out.reshape(B, S, D)


# speedup vs baseline: 1.0320x; 1.0320x over previous
import jax
import jax.numpy as jnp
from jax.experimental import pallas as pl
from jax.experimental.pallas import tpu as pltpu

_NCHUNK = 4
_NSUB = 2


def _mm_manual(x_hbm, w_ref, o_hbm, xbuf, obuf, insem, outsem):
    n, c = xbuf.shape[0], xbuf.shape[1]
    s = c // _NSUB
    for i in range(n):
        pltpu.make_async_copy(
            x_hbm.at[pl.ds(i * c, c), :], xbuf.at[i], insem.at[i]
        ).start()
    for i in range(n):
        pltpu.make_async_copy(
            x_hbm.at[pl.ds(i * c, c), :], xbuf.at[i], insem.at[i]
        ).wait()
        for j in range(_NSUB):
            k = i * _NSUB + j
            obuf[k] = jax.lax.dot_general(
                xbuf[i, pl.ds(j * s, s), :], w_ref[...],
                dimension_numbers=(((1,), (1,)), ((), ())),
                preferred_element_type=jnp.float32,
            )
            pltpu.make_async_copy(
                obuf.at[k], o_hbm.at[pl.ds(k * s, s), :], outsem.at[k]
            ).start()
    for k in range(n * _NSUB):
        s0 = o_hbm.shape[0] // (n * _NSUB)
        pltpu.make_async_copy(
            obuf.at[k], o_hbm.at[pl.ds(k * s0, s0), :], outsem.at[k]
        ).wait()


def kernel(x, W):
    B, S, D = x.shape
    M = B * S
    c = M // _NCHUNK
    s = c // _NSUB
    x2 = x.reshape(M, D)
    out = pl.pallas_call(
        _mm_manual,
        in_specs=[
            pl.BlockSpec(memory_space=pltpu.MemorySpace.HBM),
            pl.BlockSpec(memory_space=pltpu.MemorySpace.VMEM),
        ],
        out_specs=pl.BlockSpec(memory_space=pltpu.MemorySpace.HBM),
        out_shape=jax.ShapeDtypeStruct((M, D), jnp.float32),
        scratch_shapes=[
            pltpu.VMEM((_NCHUNK, c, D), jnp.float32),
            pltpu.VMEM((_NCHUNK * _NSUB, s, D), jnp.float32),
            pltpu.SemaphoreType.DMA((_NCHUNK,)),
            pltpu.SemaphoreType.DMA((_NCHUNK * _NSUB,)),
        ],
    )(x2, W)
    return out.reshape(B, S, D)


# speedup vs baseline: 1.0671x; 1.0340x over previous
import jax
import jax.numpy as jnp
from jax.experimental import pallas as pl
from jax.experimental.pallas import tpu as pltpu


def _rowmm_kernel(x_ref, w_ref, o_ref):
    o_ref[...] = jax.lax.dot_general(
        x_ref[...], w_ref[...],
        dimension_numbers=(((1,), (1,)), ((), ())),
        preferred_element_type=jnp.float32,
    )


def kernel(x, W):
    B, S, D = x.shape
    M = B * S
    x2 = x.reshape(M, D)
    block_m = 16384
    out = pl.pallas_call(
        _rowmm_kernel,
        grid=(M // block_m,),
        in_specs=[
            pl.BlockSpec((block_m, D), lambda i: (i, 0)),
            pl.BlockSpec((D, D), lambda i: (0, 0)),
        ],
        out_specs=pl.BlockSpec((block_m, D), lambda i: (i, 0)),
        out_shape=jax.ShapeDtypeStruct((M, D), jnp.float32),
        compiler_params=pltpu.CompilerParams(
            dimension_semantics=(pltpu.PARALLEL,),
        ),
    )(x2, W)
    return out.reshape(B, S, D)
